# Initial kernel scaffold; baseline (speedup 1.0000x reference)
import jax
import jax.numpy as jnp
from jax.experimental import pallas as pl


def kernel(x, edge_index, edge_weight, weight, bias):
    raise NotImplementedError("write your pallas kernel here")


# speedup vs baseline: 5.3124x; 5.3124x over previous
import functools

import jax
import jax.numpy as jnp
from jax import lax
from jax.experimental import pallas as pl
from jax.experimental.pallas import tpu as pltpu
from jax.experimental.pallas import tpu_sc as plsc

_MIN_NORM = 1e-15
_BALL_EPS = 4e-3
_ATANH_CLIP = 1.0 - 1e-7


def _artanh(y):
    yc = jnp.clip(y, -_ATANH_CLIP, _ATANH_CLIP)
    return 0.5 * jnp.log((1.0 + yc) / (1.0 - yc))


def _rownorm(v):
    return jnp.maximum(jnp.sqrt(jnp.sum(v * v, axis=-1, keepdims=True)), _MIN_NORM)


def _proj(v):
    n = _rownorm(v)
    maxnorm = 1.0 - _BALL_EPS
    return jnp.where(n > maxnorm, v / n * maxnorm, v)


def _expmap0(u):
    n = _rownorm(u)
    return jnp.tanh(n) * u / n


def _logmap0(p):
    n = _rownorm(p)
    return _artanh(n) * p / n


def _mobius_add(x, y):
    x2 = jnp.sum(x * x, axis=-1, keepdims=True)
    y2 = jnp.sum(y * y, axis=-1, keepdims=True)
    xy = jnp.sum(x * y, axis=-1, keepdims=True)
    num = (1.0 + 2.0 * xy + y2) * x + (1.0 - x2) * y
    den = 1.0 + 2.0 * xy + x2 * y2
    return num / jnp.maximum(den, _MIN_NORM)


def _linear_body(x_ref, w_ref, b_ref, o_ref):
    xb = x_ref[...]
    w = w_ref[...]
    mx = lax.dot_general(xb, w, (((1,), (1,)), ((), ())),
                         preferred_element_type=jnp.float32)
    x_n = _rownorm(xb)
    mx_n = _rownorm(mx)
    res = jnp.tanh(mx_n / x_n * _artanh(x_n)) * mx / mx_n
    allzero = jnp.all(mx == 0.0, axis=-1, keepdims=True)
    mv = jnp.where(allzero, jnp.zeros_like(res), res)
    res_p = _proj(mv)
    hyp_bias = _proj(_expmap0(b_ref[...]))
    h = _proj(_mobius_add(res_p, hyp_bias))
    o_ref[...] = _logmap0(h)


def _tc_linear(x, weight, bias2d):
    n, d_in = x.shape
    d_out = weight.shape[0]
    blk = 1000
    return pl.pallas_call(
        _linear_body,
        grid=(n // blk,),
        in_specs=[
            pl.BlockSpec((blk, d_in), lambda i: (i, 0)),
            pl.BlockSpec((d_out, d_in), lambda i: (0, 0)),
            pl.BlockSpec((1, d_out), lambda i: (0, 0)),
        ],
        out_specs=pl.BlockSpec((blk, d_out), lambda i: (i, 0)),
        out_shape=jax.ShapeDtypeStruct((n, d_out), jnp.float32),
    )(x, weight, bias2d)


def _splat_lane(vec16, j):
    idx = jnp.full((16, 1), j, jnp.int32)
    return lax.gather(
        vec16, idx,
        lax.GatherDimensionNumbers(offset_dims=(), collapsed_slice_dims=(0,),
                                   start_index_map=(0,)),
        (1,), mode=lax.GatherScatterMode.PROMISE_IN_BOUNDS)


def _sc_aggregate(xt, src, dst, ew):
    n, d = xt.shape
    e = src.shape[0]
    chunk = 128
    n_chunks = e // chunk
    ncores, nsub = 2, 16
    nw = ncores * nsub
    nlane = 16
    zblk = 16
    n_zblks = n // zblk
    mesh = plsc.VectorSubcoreMesh(core_axis_name="c", subcore_axis_name="s")

    @functools.partial(
        pl.kernel,
        mesh=mesh,
        out_type=jax.ShapeDtypeStruct((ncores, n, d), jnp.float32),
        scratch_types=[
            pltpu.VMEM((chunk,), jnp.int32),
            pltpu.VMEM((chunk,), jnp.int32),
            pltpu.VMEM((chunk,), jnp.float32),
            pltpu.VMEM((chunk, d), jnp.float32),
            pltpu.VMEM_SHARED((n, d), jnp.float32),
            pltpu.SemaphoreType.DMA,
        ],
    )
    def body(xt_hbm, src_hbm, dst_hbm, ew_hbm, out_hbm,
             src_v, dst_v, w_v, rows_v, acc_sh, sem):
        cid = lax.axis_index("c")
        sid = lax.axis_index("s")
        wid = sid * ncores + cid

        zeros16 = jnp.zeros((nlane,), jnp.float32)

        def zero_row(r, carry):
            for f in range(d // nlane):
                rows_v[r, pl.ds(f * nlane, nlane)] = zeros16
            return carry

        lax.fori_loop(0, chunk, zero_row, 0)

        n_my_z = (n_zblks - sid + nsub - 1) // nsub

        def zero_blk(k, carry):
            off = pl.multiple_of((sid + k * nsub) * zblk, zblk)
            pltpu.sync_copy(rows_v.at[pl.ds(0, zblk)],
                            acc_sh.at[pl.ds(off, zblk)])
            return carry

        lax.fori_loop(0, n_my_z, zero_blk, 0)
        plsc.subcore_barrier()

        n_my = (n_chunks - wid + nw - 1) // nw

        def chunk_body(k, carry):
            base = (wid + k * nw) * chunk
            pltpu.sync_copy(src_hbm.at[pl.ds(base, chunk)], src_v)
            pltpu.sync_copy(dst_hbm.at[pl.ds(base, chunk)], dst_v)
            pltpu.sync_copy(ew_hbm.at[pl.ds(base, chunk)], w_v)
            pltpu.async_copy(xt_hbm.at[src_v], rows_v, sem).wait()

            def scale_group(g, c2):
                wv = w_v[pl.ds(g * nlane, nlane)]
                for j in range(nlane):
                    we = _splat_lane(wv, j)
                    r = g * nlane + j
                    for f in range(d // nlane):
                        sl = pl.ds(f * nlane, nlane)
                        rows_v[r, sl] = rows_v[r, sl] * we
                return c2

            lax.fori_loop(0, chunk // nlane, scale_group, 0)
            pltpu.sync_copy(rows_v, acc_sh.at[dst_v], add=True)
            return carry

        lax.fori_loop(0, n_my, chunk_body, 0)
        plsc.subcore_barrier()

        def out_blk(k, carry):
            off = pl.multiple_of((sid + k * nsub) * zblk, zblk)
            pltpu.sync_copy(acc_sh.at[pl.ds(off, zblk)],
                            rows_v.at[pl.ds(0, zblk)])
            pltpu.sync_copy(rows_v.at[pl.ds(0, zblk)],
                            out_hbm.at[cid, pl.ds(off, zblk)])
            return carry

        lax.fori_loop(0, n_my_z, out_blk, 0)

    return body(xt, src, dst, ew)


def _finish_body(a_ref, o_ref):
    s = a_ref[0] + a_ref[1]
    h = _proj(_expmap0(s))
    xt = jnp.maximum(_logmap0(h), 0.0)
    o_ref[...] = _proj(_expmap0(xt))


def _tc_finish(acc):
    _, n, d = acc.shape
    blk = 1000
    return pl.pallas_call(
        _finish_body,
        grid=(n // blk,),
        in_specs=[pl.BlockSpec((2, blk, d), lambda i: (0, i, 0))],
        out_specs=pl.BlockSpec((blk, d), lambda i: (i, 0)),
        out_shape=jax.ShapeDtypeStruct((n, d), jnp.float32),
    )(acc)


def kernel(x, edge_index, edge_weight, weight, bias):
    xt = _tc_linear(x, weight, bias.reshape(1, -1))
    acc = _sc_aggregate(xt, edge_index[0], edge_index[1], edge_weight)
    return _tc_finish(acc)
